# baseline (speedup 1.0000x reference)
import functools
import math

import jax
import jax.numpy as jnp
from jax import lax
from jax.experimental import pallas as pl
from jax.experimental.pallas import tpu as pltpu
from jax.experimental.pallas import tpu_sc as plsc

VOCAB = 100000
D = 64
MAX_LEN = 1024
LANES = 16
NC, NS = 2, 16
NW = NC * NS
ROWS = 1024 * 1024
ROWS_PER_W = ROWS // NW
CHUNK = 128
N_CHUNKS = ROWS_PER_W // CHUNK
IDX_HALF = N_CHUNKS // 2
ST = MAX_LEN // CHUNK
DT = D // 8


def _sin_pe(max_len, d_model):
    pos = jnp.arange(0, max_len, dtype=jnp.float32)[:, None]
    div = jnp.exp(jnp.arange(0, d_model, 2, dtype=jnp.float32)
                  * (-(math.log(10000.0) / d_model)))
    pe = jnp.zeros((max_len, d_model), dtype=jnp.float32)
    pe = pe.at[:, 0::2].set(jnp.sin(pos * div))
    pe = pe.at[:, 1::2].set(jnp.cos(pos * div))
    return pe


def _sc_body(table_hbm, ids_hbm, pe_hbm, out_hbm,
             idx_v, rows0, rows1, tb0, tb1, pet_v, gsem, osem):
    wid = lax.axis_index("s") * NC + lax.axis_index("c")
    c0 = wid * N_CHUNKS
    pltpu.sync_copy(ids_hbm.at[pl.ds(c0, IDX_HALF)], idx_v)
    pltpu.sync_copy(pe_hbm, pet_v)

    iot = lax.iota(jnp.int32, LANES)
    fd = [iot + (LANES * j4) for j4 in range(D // LANES)]

    def g_issue(c, buf):
        pltpu.async_copy(table_hbm.at[idx_v.at[lax.rem(c, IDX_HALF)]],
                         buf, gsem)

    def g_wait(buf):
        pltpu.make_async_copy(table_hbm.at[idx_v.at[0]], buf, gsem).wait()

    def s_issue(c, tb):
        g = c0 + c
        b, st = g // ST, lax.rem(g, ST)
        for dt in range(DT):
            pltpu.async_copy(tb.at[pl.ds(8 * dt, 8), pl.ds(0, CHUNK)],
                             out_hbm.at[b, dt, st], osem)

    def s_wait(tb):
        for dt in range(DT):
            pltpu.make_async_copy(tb.at[pl.ds(0, 8), pl.ds(0, CHUNK)],
                                  out_hbm.at[0, 0, 0], osem).wait()

    def trans_add(c, rows, tb):
        sb = lax.rem(c0 + c, ST) * CHUNK

        def sloop(sc, carry):
            scv = jnp.full((LANES,), sc, jnp.int32)
            for j4 in range(D // LANES):
                ds = pl.ds(LANES * j4, LANES)
                v = rows[sc, ds] + pet_v[sb + sc, ds]
                plsc.store_scatter(tb, [fd[j4], scv], v)
            return carry

        lax.fori_loop(0, CHUNK, sloop, 0, unroll=4)

    g_issue(0, rows0)
    g_wait(rows0)
    g_issue(1, rows1)
    trans_add(0, rows0, tb0)
    s_issue(0, tb0)
    g_wait(rows1)
    g_issue(2, rows0)
    trans_add(1, rows1, tb1)
    s_issue(1, tb1)

    def body(C, carry):
        e = 2 * C
        g_wait(rows0)
        s_wait(tb0)
        g_issue(e + 1, rows1)
        trans_add(e, rows0, tb0)
        s_issue(e, tb0)
        g_wait(rows1)
        s_wait(tb1)

        @pl.when(C == IDX_HALF // 2 - 1)
        def _():
            pltpu.sync_copy(ids_hbm.at[pl.ds(c0 + IDX_HALF, IDX_HALF)],
                            idx_v)

        @pl.when(C < N_CHUNKS // 2 - 1)
        def _():
            g_issue(e + 2, rows0)

        trans_add(e + 1, rows1, tb1)
        s_issue(e + 1, tb1)
        return carry

    lax.fori_loop(1, N_CHUNKS // 2, body, 0)
    s_wait(tb0)
    s_wait(tb1)


@jax.jit
def _tpe_sc(ids_flat2d, table, pe_t):
    mesh = plsc.VectorSubcoreMesh(core_axis_name="c", subcore_axis_name="s")
    k = functools.partial(
        pl.kernel,
        out_type=jax.ShapeDtypeStruct((ROWS // MAX_LEN, DT, ST, 8, CHUNK),
                                      jnp.float32),
        mesh=mesh,
        scratch_types=[
            pltpu.VMEM((IDX_HALF, CHUNK), jnp.int32),
            pltpu.VMEM((CHUNK, D), jnp.float32),
            pltpu.VMEM((CHUNK, D), jnp.float32),
            pltpu.VMEM((D, CHUNK + 5), jnp.float32),
            pltpu.VMEM((D, CHUNK + 5), jnp.float32),
            pltpu.VMEM((MAX_LEN, D), jnp.float32),
            pltpu.SemaphoreType.DMA,
            pltpu.SemaphoreType.DMA,
        ],
        compiler_params=pltpu.CompilerParams(use_tc_tiling_on_sc=False,
                                             needs_layout_passes=False),
    )(_sc_body)
    return k(table, ids_flat2d, pe_t)


def kernel(input_ids, table):
    ids = input_ids.reshape(ROWS // CHUNK, CHUNK).astype(jnp.int32)
    pe = _sin_pe(MAX_LEN, D)
    out5 = _tpe_sc(ids, table, pe)
    return out5.transpose(0, 2, 4, 1, 3).reshape(ROWS // MAX_LEN, MAX_LEN, D)


# speedup vs baseline: 1.0031x; 1.0031x over previous
import functools
import math

import jax
import jax.numpy as jnp
from jax import lax
from jax.experimental import pallas as pl
from jax.experimental.pallas import tpu as pltpu
from jax.experimental.pallas import tpu_sc as plsc

VOCAB = 100000
D = 64
MAX_LEN = 1024
LANES = 16
NC, NS = 2, 16
NW = NC * NS
ROWS = 1024 * 1024
ROWS_PER_W = ROWS // NW
CHUNK = 128
N_CHUNKS = ROWS_PER_W // CHUNK
IDX_HALF = N_CHUNKS // 2
ST = MAX_LEN // CHUNK
DT = D // 8


def _sin_pe(max_len, d_model):
    pos = jnp.arange(0, max_len, dtype=jnp.float32)[:, None]
    div = jnp.exp(jnp.arange(0, d_model, 2, dtype=jnp.float32)
                  * (-(math.log(10000.0) / d_model)))
    pe = jnp.zeros((max_len, d_model), dtype=jnp.float32)
    pe = pe.at[:, 0::2].set(jnp.sin(pos * div))
    pe = pe.at[:, 1::2].set(jnp.cos(pos * div))
    return pe


def _sc_body(table_hbm, ids_hbm, pe_hbm, out_hbm,
             idx_v, rows0, rows1, tb0, tb1, pet_v, gsem, osem):
    wid = lax.axis_index("s") * NC + lax.axis_index("c")
    c0 = wid * N_CHUNKS
    pltpu.sync_copy(ids_hbm.at[pl.ds(c0, IDX_HALF)], idx_v)
    pltpu.sync_copy(pe_hbm, pet_v)

    iot = lax.iota(jnp.int32, LANES)
    fd = [iot + (LANES * j4) for j4 in range(D // LANES)]

    def g_issue(c, buf):
        pltpu.async_copy(table_hbm.at[idx_v.at[lax.rem(c, IDX_HALF)]],
                         buf, gsem)

    def g_wait(buf):
        pltpu.make_async_copy(table_hbm.at[idx_v.at[0]], buf, gsem).wait()

    def s_issue(c, tb):
        g = c0 + c
        b, st = g // ST, lax.rem(g, ST)
        for dt in range(DT):
            pltpu.async_copy(tb.at[pl.ds(8 * dt, 8), pl.ds(0, CHUNK)],
                             out_hbm.at[b, dt, st], osem)

    def s_wait(tb):
        for dt in range(DT):
            pltpu.make_async_copy(tb.at[pl.ds(0, 8), pl.ds(0, CHUNK)],
                                  out_hbm.at[0, 0, 0], osem).wait()

    def trans_add(c, rows, tb):
        sb = lax.rem(c0 + c, ST) * CHUNK

        def sloop(sc, scv):
            for j4 in range(D // LANES):
                ds = pl.ds(LANES * j4, LANES)
                v = rows[sc, ds] + pet_v[sb + sc, ds]
                plsc.store_scatter(tb, [fd[j4], scv], v)
            return scv + 1

        lax.fori_loop(0, CHUNK, sloop, jnp.zeros((LANES,), jnp.int32),
                      unroll=4)

    g_issue(0, rows0)
    g_wait(rows0)
    g_issue(1, rows1)
    trans_add(0, rows0, tb0)
    s_issue(0, tb0)
    g_wait(rows1)
    g_issue(2, rows0)
    trans_add(1, rows1, tb1)
    s_issue(1, tb1)

    def body(C, carry):
        e = 2 * C
        g_wait(rows0)
        s_wait(tb0)
        g_issue(e + 1, rows1)
        trans_add(e, rows0, tb0)
        s_issue(e, tb0)
        g_wait(rows1)
        s_wait(tb1)

        @pl.when(C == IDX_HALF // 2 - 1)
        def _():
            pltpu.sync_copy(ids_hbm.at[pl.ds(c0 + IDX_HALF, IDX_HALF)],
                            idx_v)

        @pl.when(C < N_CHUNKS // 2 - 1)
        def _():
            g_issue(e + 2, rows0)

        trans_add(e + 1, rows1, tb1)
        s_issue(e + 1, tb1)
        return carry

    lax.fori_loop(1, N_CHUNKS // 2, body, 0)
    s_wait(tb0)
    s_wait(tb1)


@jax.jit
def _tpe_sc(ids_flat2d, table, pe_t):
    mesh = plsc.VectorSubcoreMesh(core_axis_name="c", subcore_axis_name="s")
    k = functools.partial(
        pl.kernel,
        out_type=jax.ShapeDtypeStruct((ROWS // MAX_LEN, DT, ST, 8, CHUNK),
                                      jnp.float32),
        mesh=mesh,
        scratch_types=[
            pltpu.VMEM((IDX_HALF, CHUNK), jnp.int32),
            pltpu.VMEM((CHUNK, D), jnp.float32),
            pltpu.VMEM((CHUNK, D), jnp.float32),
            pltpu.VMEM((D, CHUNK + 5), jnp.float32),
            pltpu.VMEM((D, CHUNK + 5), jnp.float32),
            pltpu.VMEM((MAX_LEN, D), jnp.float32),
            pltpu.SemaphoreType.DMA,
            pltpu.SemaphoreType.DMA,
        ],
        compiler_params=pltpu.CompilerParams(use_tc_tiling_on_sc=False,
                                             needs_layout_passes=False),
    )(_sc_body)
    return k(table, ids_flat2d, pe_t)


def kernel(input_ids, table):
    ids = input_ids.reshape(ROWS // CHUNK, CHUNK).astype(jnp.int32)
    pe = _sin_pe(MAX_LEN, D)
    out5 = _tpe_sc(ids, table, pe)
    return out5.transpose(0, 2, 4, 1, 3).reshape(ROWS // MAX_LEN, MAX_LEN, D)
